# Initial kernel scaffold; baseline (speedup 1.0000x reference)
import jax
import jax.numpy as jnp
from jax.experimental import pallas as pl


def kernel(atomic_numbers, charge_table):
    raise NotImplementedError("write your pallas kernel here")


# speedup vs baseline: 239.8554x; 239.8554x over previous
import functools

import jax
import jax.numpy as jnp
from jax import lax
from jax.experimental import pallas as pl
from jax.experimental.pallas import tpu as pltpu
from jax.experimental.pallas import tpu_sc as plsc

_N = 2_000_000
_NC = 2
_NS = 16
_NW = _NC * _NS
_L = 16
_CHUNK = 12_512
_NCHUNK = 5
_PER_TILE = _CHUNK * _NCHUNK
_N_PAD = _PER_TILE * _NW
_TBL_PAD = 128

_NORM = 0.5

_mesh = plsc.VectorSubcoreMesh(core_axis_name="c", subcore_axis_name="s")


@functools.partial(
    pl.kernel,
    mesh=_mesh,
    compiler_params=pltpu.CompilerParams(needs_layout_passes=False),
    out_type=jax.ShapeDtypeStruct((_N_PAD,), jnp.float32),
    scratch_types=[
        pltpu.VMEM((_TBL_PAD,), jnp.float32),
        pltpu.VMEM((_CHUNK,), jnp.int32),
        pltpu.VMEM((_CHUNK,), jnp.float32),
    ],
)
def _fixed_charges_sc(idx_hbm, tbl_hbm, out_hbm, tbl_v, idx_v, out_v):
    wid = lax.axis_index("s") * _NC + lax.axis_index("c")
    base = wid * _PER_TILE

    pltpu.sync_copy(tbl_hbm, tbl_v)
    for i in range(_TBL_PAD // _L):
        sl = pl.ds(i * _L, _L)
        tbl_v[sl] = tbl_v[sl] * jnp.float32(_NORM)

    for g in range(_NCHUNK):
        off = base + g * _CHUNK

        pltpu.sync_copy(idx_hbm.at[pl.ds(off, _CHUNK)], idx_v)

        def body(j, carry):
            sl = pl.ds(j * _L, _L)
            out_v[sl] = plsc.load_gather(tbl_v, [idx_v[sl]])
            return carry

        lax.fori_loop(0, _CHUNK // _L, body, 0)

        pltpu.sync_copy(out_v, out_hbm.at[pl.ds(off, _CHUNK)])


def kernel(atomic_numbers, charge_table):
    idx = jnp.pad(atomic_numbers.astype(jnp.int32), (0, _N_PAD - _N))
    tbl = jnp.pad(
        charge_table.astype(jnp.float32),
        (0, _TBL_PAD - charge_table.shape[0]),
    )
    out = _fixed_charges_sc(idx, tbl)
    return out[:_N]


# speedup vs baseline: 361.8444x; 1.5086x over previous
import functools

import jax
import jax.numpy as jnp
from jax import lax
from jax.experimental import pallas as pl
from jax.experimental.pallas import tpu as pltpu
from jax.experimental.pallas import tpu_sc as plsc

_N = 2_000_000
_NC = 2
_NS = 16
_NW = _NC * _NS
_L = 16
_CHUNK = 6_272
_NCHUNK = 10
_PER_TILE = _CHUNK * _NCHUNK
_N_PAD = _PER_TILE * _NW
_TBL_PAD = 128

_NORM = 0.5

_mesh = plsc.VectorSubcoreMesh(core_axis_name="c", subcore_axis_name="s")


@functools.partial(
    pl.kernel,
    mesh=_mesh,
    compiler_params=pltpu.CompilerParams(needs_layout_passes=False),
    out_type=jax.ShapeDtypeStruct((_N_PAD,), jnp.float32),
    scratch_types=[
        pltpu.VMEM((_TBL_PAD,), jnp.float32),
        pltpu.VMEM((_CHUNK,), jnp.int32),
        pltpu.VMEM((_CHUNK,), jnp.int32),
        pltpu.VMEM((_CHUNK,), jnp.float32),
        pltpu.VMEM((_CHUNK,), jnp.float32),
        pltpu.SemaphoreType.DMA,
        pltpu.SemaphoreType.DMA,
        pltpu.SemaphoreType.DMA,
        pltpu.SemaphoreType.DMA,
    ],
)
def _fixed_charges_sc(idx_hbm, tbl_hbm, out_hbm, tbl_v, idx_v0, idx_v1,
                      out_v0, out_v1, sem_i0, sem_i1, sem_o0, sem_o1):
    wid = lax.axis_index("s") * _NC + lax.axis_index("c")
    base = wid * _PER_TILE

    idx_bufs = (idx_v0, idx_v1)
    out_bufs = (out_v0, out_v1)
    sems_in = (sem_i0, sem_i1)
    sems_out = (sem_o0, sem_o1)
    copies_in = [None, None]
    copies_out = [None, None]

    def start_in(g):
        b = g % 2
        off = base + g * _CHUNK
        copies_in[b] = pltpu.async_copy(
            idx_hbm.at[pl.ds(off, _CHUNK)], idx_bufs[b], sems_in[b]
        )

    start_in(0)

    pltpu.sync_copy(tbl_hbm, tbl_v)
    for i in range(_TBL_PAD // _L):
        sl = pl.ds(i * _L, _L)
        tbl_v[sl] = tbl_v[sl] * jnp.float32(_NORM)

    for g in range(_NCHUNK):
        b = g % 2
        copies_in[b].wait()
        if g + 1 < _NCHUNK:
            start_in(g + 1)
        if copies_out[b] is not None:
            copies_out[b].wait()

        ib = idx_bufs[b]
        ob = out_bufs[b]

        @plsc.parallel_loop(0, _CHUNK, step=_L, unroll=8)
        def _gather(i):
            ob[pl.ds(i, _L)] = plsc.load_gather(tbl_v, [ib[pl.ds(i, _L)]])

        off = base + g * _CHUNK
        copies_out[b] = pltpu.async_copy(
            out_bufs[b], out_hbm.at[pl.ds(off, _CHUNK)], sems_out[b]
        )

    copies_out[0].wait()
    copies_out[1].wait()


def kernel(atomic_numbers, charge_table):
    idx = jnp.pad(atomic_numbers.astype(jnp.int32), (0, _N_PAD - _N))
    tbl = jnp.pad(
        charge_table.astype(jnp.float32),
        (0, _TBL_PAD - charge_table.shape[0]),
    )
    out = _fixed_charges_sc(idx, tbl)
    return out[:_N]


# speedup vs baseline: 488.5219x; 1.3501x over previous
import functools

import jax
import jax.numpy as jnp
from jax import lax
from jax.experimental import pallas as pl
from jax.experimental.pallas import tpu as pltpu
from jax.experimental.pallas import tpu_sc as plsc

_N = 2_000_000
_NC = 2
_NS = 16
_NW = _NC * _NS
_L = 16
_ROW = 128
_CHUNK = 61 * _ROW
_NCHUNK = 8
_PER_TILE = _CHUNK * _NCHUNK
_TAIL_ROW0 = _NW * _PER_TILE // _ROW
_TBL_PAD = 128

_NORM = 0.5

_mesh = plsc.VectorSubcoreMesh(core_axis_name="c", subcore_axis_name="s")


@functools.partial(
    pl.kernel,
    mesh=_mesh,
    compiler_params=pltpu.CompilerParams(needs_layout_passes=False),
    out_type=jax.ShapeDtypeStruct((_N,), jnp.float32),
    scratch_types=[
        pltpu.VMEM((_TBL_PAD,), jnp.float32),
        pltpu.VMEM((_CHUNK,), jnp.int32),
        pltpu.VMEM((_CHUNK,), jnp.int32),
        pltpu.VMEM((_CHUNK,), jnp.float32),
        pltpu.VMEM((_CHUNK,), jnp.float32),
        pltpu.SemaphoreType.DMA,
        pltpu.SemaphoreType.DMA,
        pltpu.SemaphoreType.DMA,
        pltpu.SemaphoreType.DMA,
    ],
)
def _fixed_charges_sc(idx_hbm, tbl_hbm, out_hbm, tbl_v, idx_v0, idx_v1,
                      out_v0, out_v1, sem_i0, sem_i1, sem_o0, sem_o1):
    wid = lax.axis_index("s") * _NC + lax.axis_index("c")
    base = wid * _PER_TILE

    idx_bufs = (idx_v0, idx_v1)
    out_bufs = (out_v0, out_v1)
    sems_in = (sem_i0, sem_i1)
    sems_out = (sem_o0, sem_o1)
    copies_in = [None, None]
    copies_out = [None, None]

    def start_in(g):
        b = g % 2
        off = base + g * _CHUNK
        copies_in[b] = pltpu.async_copy(
            idx_hbm.at[pl.ds(off, _CHUNK)], idx_bufs[b], sems_in[b]
        )

    start_in(0)

    pltpu.sync_copy(tbl_hbm, tbl_v)
    for i in range(_TBL_PAD // _L):
        sl = pl.ds(i * _L, _L)
        tbl_v[sl] = tbl_v[sl] * jnp.float32(_NORM)

    @pl.when(wid < _N // _ROW - _TAIL_ROW0)
    def _tail():
        off = (_TAIL_ROW0 + wid) * _ROW
        pltpu.sync_copy(idx_hbm.at[pl.ds(off, _ROW)], idx_v1.at[pl.ds(0, _ROW)])
        for i in range(_ROW // _L):
            sl = pl.ds(i * _L, _L)
            out_v1[sl] = plsc.load_gather(tbl_v, [idx_v1[sl]])
        pltpu.sync_copy(out_v1.at[pl.ds(0, _ROW)], out_hbm.at[pl.ds(off, _ROW)])

    for g in range(_NCHUNK):
        b = g % 2
        copies_in[b].wait()
        if g + 1 < _NCHUNK:
            start_in(g + 1)
        if copies_out[b] is not None:
            copies_out[b].wait()

        ib = idx_bufs[b]
        ob = out_bufs[b]

        @plsc.parallel_loop(0, _CHUNK, step=_L, unroll=8)
        def _gather(i):
            ob[pl.ds(i, _L)] = plsc.load_gather(tbl_v, [ib[pl.ds(i, _L)]])

        off = base + g * _CHUNK
        copies_out[b] = pltpu.async_copy(
            out_bufs[b], out_hbm.at[pl.ds(off, _CHUNK)], sems_out[b]
        )

    copies_out[0].wait()
    copies_out[1].wait()


def kernel(atomic_numbers, charge_table):
    idx = atomic_numbers.astype(jnp.int32)
    tbl = jnp.pad(
        charge_table.astype(jnp.float32),
        (0, _TBL_PAD - charge_table.shape[0]),
    )
    return _fixed_charges_sc(idx, tbl)


# speedup vs baseline: 524.8975x; 1.0745x over previous
import functools

import jax
import jax.numpy as jnp
from jax import lax
from jax.experimental import pallas as pl
from jax.experimental.pallas import tpu as pltpu
from jax.experimental.pallas import tpu_sc as plsc

_N = 2_000_000
_NC = 2
_NS = 16
_NW = _NC * _NS
_L = 16
_ROW = 128
_CHUNK = 61 * _ROW
_NCHUNK = 8
_PER_TILE = _CHUNK * _NCHUNK
_TAIL_ROW0 = _NW * _PER_TILE // _ROW
_NTAIL = _N // _ROW - _TAIL_ROW0
_TBL = 119
_TBL_PAD = 128

_NORM = 0.5

_mesh = plsc.VectorSubcoreMesh(core_axis_name="c", subcore_axis_name="s")


@functools.partial(
    pl.kernel,
    mesh=_mesh,
    compiler_params=pltpu.CompilerParams(needs_layout_passes=False),
    out_type=jax.ShapeDtypeStruct((_N,), jnp.float32),
    scratch_types=[
        pltpu.VMEM((_TBL_PAD,), jnp.float32),
        pltpu.VMEM((_CHUNK,), jnp.int32),
        pltpu.VMEM((_CHUNK,), jnp.int32),
        pltpu.VMEM((_CHUNK,), jnp.float32),
        pltpu.VMEM((_CHUNK,), jnp.float32),
        pltpu.VMEM((_TBL_PAD,), jnp.float32),
        pltpu.VMEM((_ROW,), jnp.int32),
        pltpu.VMEM((_ROW,), jnp.float32),
        pltpu.SemaphoreType.DMA,
        pltpu.SemaphoreType.DMA,
        pltpu.SemaphoreType.DMA,
        pltpu.SemaphoreType.DMA,
    ],
)
def _fixed_charges_sc(idx_hbm, tbl_hbm, out_hbm, tbl_v, idx_v0, idx_v1,
                      out_v0, out_v1, tbl_raw, tidx_v, tout_v,
                      sem_i0, sem_i1, sem_o0, sem_o1):
    wid = lax.axis_index("s") * _NC + lax.axis_index("c")
    base = wid * _PER_TILE

    idx_bufs = (idx_v0, idx_v1)
    out_bufs = (out_v0, out_v1)
    sems_in = (sem_i0, sem_i1)
    sems_out = (sem_o0, sem_o1)

    for b in range(2):
        pltpu.async_copy(
            idx_hbm.at[pl.ds(base + b * _CHUNK, _CHUNK)], idx_bufs[b],
            sems_in[b],
        )

    pltpu.sync_copy(tbl_hbm, tbl_raw.at[pl.ds(0, _TBL)])
    for s in list(range(0, _TBL - _L, _L)) + [_TBL - _L]:
        sl = pl.ds(s, _L)
        tbl_v[sl] = tbl_raw[sl] * jnp.float32(_NORM)

    @pl.when(wid < _NTAIL)
    def _tail():
        off = (_TAIL_ROW0 + wid) * _ROW
        pltpu.sync_copy(idx_hbm.at[pl.ds(off, _ROW)], tidx_v)
        for i in range(_ROW // _L):
            sl = pl.ds(i * _L, _L)
            tout_v[sl] = plsc.load_gather(tbl_v, [tidx_v[sl]])
        pltpu.sync_copy(tout_v, out_hbm.at[pl.ds(off, _ROW)])

    @pl.loop(0, _NCHUNK, step=2)
    def _ring(g0):
        for b in range(2):
            gg = g0 + b
            off = base + gg * _CHUNK
            ib = idx_bufs[b]
            ob = out_bufs[b]

            pltpu.make_async_copy(
                idx_hbm.at[pl.ds(off, _CHUNK)], ib, sems_in[b]
            ).wait()

            @pl.when(gg >= 2)
            def _wait_prev_out():
                pltpu.make_async_copy(
                    ob, out_hbm.at[pl.ds(off, _CHUNK)], sems_out[b]
                ).wait()

            @plsc.parallel_loop(0, _CHUNK, step=_L, unroll=8)
            def _gather(i):
                ob[pl.ds(i, _L)] = plsc.load_gather(tbl_v, [ib[pl.ds(i, _L)]])

            pltpu.async_copy(ob, out_hbm.at[pl.ds(off, _CHUNK)], sems_out[b])

            @pl.when(gg + 2 < _NCHUNK)
            def _next_in():
                off2 = base + (gg + 2) * _CHUNK
                pltpu.async_copy(
                    idx_hbm.at[pl.ds(off2, _CHUNK)], ib, sems_in[b]
                )

    for b in range(2):
        pltpu.make_async_copy(
            out_bufs[b], out_hbm.at[pl.ds(base, _CHUNK)], sems_out[b]
        ).wait()


def kernel(atomic_numbers, charge_table):
    return _fixed_charges_sc(
        atomic_numbers.astype(jnp.int32), charge_table.astype(jnp.float32)
    )
